# Initial kernel scaffold; baseline (speedup 1.0000x reference)
import jax
import jax.numpy as jnp
from jax.experimental import pallas as pl


def kernel(x, edge_index, edge_weight, weight_own, weight_nbr, weight_temp, bias):
    raise NotImplementedError("write your pallas kernel here")


# speedup vs baseline: 7.2266x; 7.2266x over previous
import functools

import jax
import jax.numpy as jnp
from jax import lax
from jax.experimental import pallas as pl
from jax.experimental.pallas import tpu as pltpu
from jax.experimental.pallas import tpu_sc as plsc

N = 10000
E = 320000
D = 128

NC = 2
NS = 16
NW = NC * NS
EPW = E // NW
CHUNK = 80
NCHUNK = EPW // CHUNK
NPAD = 10240
RPT = NPAD // NS
ZROWS = 128

ROW_BLOCK = 2000


def _matmul_body(x_ref, wo_ref, wn_ref, wt_ref, o_ref):
    w = wo_ref[...] + wn_ref[...] + wt_ref[...]
    o_ref[...] = lax.dot_general(
        x_ref[...], w, (((1,), (0,)), ((), ())),
        preferred_element_type=jnp.float32,
        precision=lax.Precision.HIGHEST,
    )


def _combine_body(p0_ref, p1_ref, b_ref, o_ref):
    o_ref[...] = p0_ref[...] + p1_ref[...] + b_ref[...]


@functools.cache
def _make_spmm_kernel():
    mesh = plsc.VectorSubcoreMesh(
        core_axis_name="c", subcore_axis_name="s",
        num_cores=NC, num_subcores=NS)
    return pl.kernel(
        _spmm_body,
        out_type=jax.ShapeDtypeStruct((NC, NPAD, D), jnp.float32),
        mesh=mesh,
        scratch_types=[
            pltpu.VMEM((CHUNK,), jnp.int32),
            pltpu.VMEM((CHUNK,), jnp.int32),
            pltpu.VMEM((CHUNK, 16), jnp.float32),
            pltpu.VMEM((CHUNK, D), jnp.float32),
            pltpu.VMEM((ZROWS, D), jnp.float32),
            pltpu.VMEM_SHARED((NPAD, D), jnp.float32),
            pltpu.SemaphoreType.DMA,
        ],
    )


def _spmm_body(s_hbm, col_hbm, row_hbm, ewb_hbm, out_hbm,
               col_v, row_v, ewb_v, rows_v, zero_v, acc_sh, sem):
    c = lax.axis_index("c")
    s = lax.axis_index("s")
    wid = s * NC + c

    def zero_body(i, carry):
        for j in range(D // 16):
            zero_v[i, pl.ds(j * 16, 16)] = jnp.zeros((16,), jnp.float32)
        return carry

    lax.fori_loop(0, ZROWS, zero_body, 0)
    for k in range(RPT // ZROWS):
        pltpu.sync_copy(zero_v, acc_sh.at[pl.ds(s * RPT + k * ZROWS, ZROWS)])
    plsc.subcore_barrier()

    base_e = wid * EPW

    def chunk_body(ch, carry):
        eoff = base_e + ch * CHUNK
        pltpu.sync_copy(col_hbm.at[pl.ds(eoff, CHUNK)], col_v)
        pltpu.sync_copy(row_hbm.at[pl.ds(eoff, CHUNK)], row_v)
        pltpu.sync_copy(ewb_hbm.at[pl.ds(eoff, CHUNK)], ewb_v)
        pltpu.async_copy(s_hbm.at[col_v], rows_v, sem).wait()

        def edge_body(i, inner):
            w = ewb_v[i, :]
            for j in range(D // 16):
                sl = pl.ds(j * 16, 16)
                rows_v[i, sl] = rows_v[i, sl] * w
            return inner

        lax.fori_loop(0, CHUNK, edge_body, 0)
        pltpu.sync_copy(rows_v, acc_sh.at[row_v], add=True)
        return carry

    lax.fori_loop(0, NCHUNK, chunk_body, 0)
    plsc.subcore_barrier()

    pltpu.sync_copy(acc_sh.at[pl.ds(s * RPT, RPT)],
                    out_hbm.at[c, pl.ds(s * RPT, RPT)])


def kernel(x, edge_index, edge_weight, weight_own, weight_nbr, weight_temp,
           bias):
    support = pl.pallas_call(
        _matmul_body,
        out_shape=jax.ShapeDtypeStruct((N, D), jnp.float32),
        grid=(N // ROW_BLOCK,),
        in_specs=[
            pl.BlockSpec((ROW_BLOCK, D), lambda i: (i, 0)),
            pl.BlockSpec((D, D), lambda i: (0, 0)),
            pl.BlockSpec((D, D), lambda i: (0, 0)),
            pl.BlockSpec((D, D), lambda i: (0, 0)),
        ],
        out_specs=pl.BlockSpec((ROW_BLOCK, D), lambda i: (i, 0)),
    )(x, weight_own, weight_nbr, weight_temp)

    row = edge_index[0].astype(jnp.int32)
    col = edge_index[1].astype(jnp.int32)
    ewb = jnp.broadcast_to(edge_weight.astype(jnp.float32)[:, None], (E, 16))

    partials = _make_spmm_kernel()(support, col, row, ewb)

    out = pl.pallas_call(
        _combine_body,
        out_shape=jax.ShapeDtypeStruct((N, D), jnp.float32),
        grid=(N // ROW_BLOCK,),
        in_specs=[
            pl.BlockSpec((ROW_BLOCK, D), lambda i: (i, 0)),
            pl.BlockSpec((ROW_BLOCK, D), lambda i: (i, 0)),
            pl.BlockSpec((1, D), lambda i: (0, 0)),
        ],
        out_specs=pl.BlockSpec((ROW_BLOCK, D), lambda i: (i, 0)),
    )(partials[0], partials[1], bias.reshape(1, D))
    return out


# speedup vs baseline: 11.0911x; 1.5348x over previous
import functools

import jax
import jax.numpy as jnp
from jax import lax
from jax.experimental import pallas as pl
from jax.experimental.pallas import tpu as pltpu
from jax.experimental.pallas import tpu_sc as plsc

N = 10000
E = 320000
D = 128

NC = 2
NS = 16
NW = NC * NS
EPW = E // NW
CHUNK = 80
NCHUNK = EPW // CHUNK
NPAD = 10240
RPT = NPAD // NS
ZROWS = 16

ROW_BLOCK = 2000


def _matmul_body(x_ref, wo_ref, wn_ref, wt_ref, o_ref):
    w = wo_ref[...] + wn_ref[...] + wt_ref[...]
    o_ref[...] = lax.dot_general(
        x_ref[...], w, (((1,), (0,)), ((), ())),
        preferred_element_type=jnp.float32,
        precision=lax.Precision.HIGHEST,
    )


def _combine_body(p0_ref, p1_ref, b_ref, o_ref):
    o_ref[...] = p0_ref[...] + p1_ref[...] + b_ref[...]


@functools.cache
def _make_spmm_kernel():
    mesh = plsc.VectorSubcoreMesh(
        core_axis_name="c", subcore_axis_name="s",
        num_cores=NC, num_subcores=NS)
    return pl.kernel(
        _spmm_body,
        out_type=jax.ShapeDtypeStruct((NC, NPAD, D), jnp.float32),
        mesh=mesh,
        scratch_types=[
            pltpu.VMEM((2, CHUNK), jnp.int32),
            pltpu.VMEM((2, CHUNK), jnp.int32),
            pltpu.VMEM((2, CHUNK, 16), jnp.float32),
            pltpu.VMEM((2, CHUNK, D), jnp.float32),
            pltpu.VMEM((ZROWS, D), jnp.float32),
            pltpu.VMEM_SHARED((NPAD, D), jnp.float32),
            pltpu.SemaphoreType.DMA,
            pltpu.SemaphoreType.DMA,
            pltpu.SemaphoreType.DMA,
            pltpu.SemaphoreType.DMA,
        ],
    )


def _spmm_body(s_hbm, col_hbm, row_hbm, ewb_hbm, out_hbm,
               col_v, row_v, ewb_v, rows_v, zero_v, acc_sh,
               gsem0, gsem1, isem0, isem1):
    gsem = (gsem0, gsem1)
    isem = (isem0, isem1)
    c = lax.axis_index("c")
    s = lax.axis_index("s")
    wid = s * NC + c

    def zero_body(i, carry):
        for j in range(D // 16):
            zero_v[i, pl.ds(j * 16, 16)] = jnp.zeros((16,), jnp.float32)
        return carry

    lax.fori_loop(0, ZROWS, zero_body, 0)
    for k in range(RPT // ZROWS):
        pltpu.sync_copy(zero_v, acc_sh.at[pl.ds(s * RPT + k * ZROWS, ZROWS)])
    plsc.subcore_barrier()

    base_e = wid * EPW

    def idx_copies(ch, b):
        eoff = base_e + ch * CHUNK
        return (
            pltpu.async_copy(col_hbm.at[pl.ds(eoff, CHUNK)], col_v.at[b],
                             isem[b]),
            pltpu.async_copy(row_hbm.at[pl.ds(eoff, CHUNK)], row_v.at[b],
                             isem[b]),
            pltpu.async_copy(ewb_hbm.at[pl.ds(eoff, CHUNK)], ewb_v.at[b],
                             isem[b]),
        )

    def wait_idx(ch, b):
        eoff = base_e + ch * CHUNK
        pltpu.make_async_copy(col_hbm.at[pl.ds(eoff, CHUNK)], col_v.at[b],
                              isem[b]).wait()
        pltpu.make_async_copy(row_hbm.at[pl.ds(eoff, CHUNK)], row_v.at[b],
                              isem[b]).wait()
        pltpu.make_async_copy(ewb_hbm.at[pl.ds(eoff, CHUNK)], ewb_v.at[b],
                              isem[b]).wait()

    def issue_gather(b):
        pltpu.async_copy(s_hbm.at[col_v.at[b]], rows_v.at[b], gsem[b])

    def wait_gather(b):
        pltpu.make_async_copy(s_hbm.at[col_v.at[b]], rows_v.at[b],
                              gsem[b]).wait()

    for b in (0, 1):
        idx_copies(b, b)
    for b in (0, 1):
        wait_idx(b, b)
        issue_gather(b)

    def chunk_step(ch, b):
        wait_gather(b)

        def edge_body(i, inner):
            w = ewb_v[b, i, :]
            for j in range(D // 16):
                sl = pl.ds(j * 16, 16)
                rows_v[b, i, sl] = rows_v[b, i, sl] * w
            return inner

        lax.fori_loop(0, CHUNK, edge_body, 0)
        pltpu.sync_copy(rows_v.at[b], acc_sh.at[row_v.at[b]], add=True)

        @pl.when(ch + 2 < NCHUNK)
        def _():
            idx_copies(ch + 2, b)
            wait_idx(ch + 2, b)
            issue_gather(b)

    def pair_body(k, carry):
        ch0 = 2 * k
        chunk_step(ch0, 0)

        @pl.when(ch0 + 1 < NCHUNK)
        def _():
            chunk_step(ch0 + 1, 1)

        return carry

    lax.fori_loop(0, (NCHUNK + 1) // 2, pair_body, 0)
    plsc.subcore_barrier()

    pltpu.sync_copy(acc_sh.at[pl.ds(s * RPT, RPT)],
                    out_hbm.at[c, pl.ds(s * RPT, RPT)])


def kernel(x, edge_index, edge_weight, weight_own, weight_nbr, weight_temp,
           bias):
    support = pl.pallas_call(
        _matmul_body,
        out_shape=jax.ShapeDtypeStruct((N, D), jnp.float32),
        grid=(N // ROW_BLOCK,),
        in_specs=[
            pl.BlockSpec((ROW_BLOCK, D), lambda i: (i, 0)),
            pl.BlockSpec((D, D), lambda i: (0, 0)),
            pl.BlockSpec((D, D), lambda i: (0, 0)),
            pl.BlockSpec((D, D), lambda i: (0, 0)),
        ],
        out_specs=pl.BlockSpec((ROW_BLOCK, D), lambda i: (i, 0)),
    )(x, weight_own, weight_nbr, weight_temp)

    row = edge_index[0].astype(jnp.int32)
    col = edge_index[1].astype(jnp.int32)
    ewb = jnp.broadcast_to(edge_weight.astype(jnp.float32)[:, None], (E, 16))

    partials = _make_spmm_kernel()(support, col, row, ewb)

    out = pl.pallas_call(
        _combine_body,
        out_shape=jax.ShapeDtypeStruct((N, D), jnp.float32),
        grid=(N // ROW_BLOCK,),
        in_specs=[
            pl.BlockSpec((ROW_BLOCK, D), lambda i: (i, 0)),
            pl.BlockSpec((ROW_BLOCK, D), lambda i: (i, 0)),
            pl.BlockSpec((1, D), lambda i: (0, 0)),
        ],
        out_specs=pl.BlockSpec((ROW_BLOCK, D), lambda i: (i, 0)),
    )(partials[0], partials[1], bias.reshape(1, D))
    return out


# speedup vs baseline: 21.7723x; 1.9630x over previous
import functools

import jax
import jax.numpy as jnp
from jax import lax
from jax.experimental import pallas as pl
from jax.experimental.pallas import tpu as pltpu
from jax.experimental.pallas import tpu_sc as plsc

N = 10000
E = 320000
D = 128

NC = 2
NS = 16
NW = NC * NS
EPW = E // NW
CHUNK = 80
NCHUNK = EPW // CHUNK
NPAD = 10240
RPT = NPAD // NS
ZROWS = 16

ROW_BLOCK = 2000


def _matmul_body(x_ref, wo_ref, wn_ref, wt_ref, o_ref):
    w = wo_ref[...] + wn_ref[...] + wt_ref[...]
    o_ref[...] = lax.dot_general(
        x_ref[...], w, (((1,), (0,)), ((), ())),
        preferred_element_type=jnp.float32,
        precision=lax.Precision.HIGHEST,
    )


def _combine_body(p0_ref, p1_ref, b_ref, o_ref):
    o_ref[...] = p0_ref[...] + p1_ref[...] + b_ref[...]


@functools.cache
def _make_spmm_kernel():
    mesh = plsc.VectorSubcoreMesh(
        core_axis_name="c", subcore_axis_name="s",
        num_cores=NC, num_subcores=NS)
    return pl.kernel(
        _spmm_body,
        out_type=jax.ShapeDtypeStruct((NC, NPAD, D), jnp.float32),
        mesh=mesh,
        scratch_types=[
            pltpu.VMEM((2, CHUNK), jnp.int32),
            pltpu.VMEM((2, CHUNK), jnp.int32),
            pltpu.VMEM((2, CHUNK), jnp.float32),
            pltpu.VMEM((2, CHUNK, D), jnp.float32),
            pltpu.VMEM((ZROWS, D), jnp.float32),
            pltpu.VMEM_SHARED((NPAD, D), jnp.float32),
            pltpu.SemaphoreType.DMA,
            pltpu.SemaphoreType.DMA,
            pltpu.SemaphoreType.DMA,
            pltpu.SemaphoreType.DMA,
        ],
    )


def _spmm_body(s_hbm, ei_hbm, ew_hbm, out_hbm,
               col_v, row_v, ew_v, rows_v, zero_v, acc_sh,
               gsem0, gsem1, isem0, isem1):
    gsem = (gsem0, gsem1)
    isem = (isem0, isem1)
    c = lax.axis_index("c")
    s = lax.axis_index("s")
    wid = s * NC + c

    def zero_body(i, carry):
        for j in range(D // 16):
            zero_v[i, pl.ds(j * 16, 16)] = jnp.zeros((16,), jnp.float32)
        return carry

    lax.fori_loop(0, ZROWS, zero_body, 0)
    for k in range(RPT // ZROWS):
        pltpu.sync_copy(zero_v, acc_sh.at[pl.ds(s * RPT + k * ZROWS, ZROWS)])
    plsc.subcore_barrier()

    base_e = wid * EPW

    def idx_copies(ch, b):
        eoff = base_e + ch * CHUNK
        pltpu.async_copy(ei_hbm.at[pl.ds(E + eoff, CHUNK)], col_v.at[b],
                         isem[b])
        pltpu.async_copy(ei_hbm.at[pl.ds(eoff, CHUNK)], row_v.at[b],
                         isem[b])
        pltpu.async_copy(ew_hbm.at[pl.ds(eoff, CHUNK)], ew_v.at[b],
                         isem[b])

    def wait_idx(ch, b):
        eoff = base_e + ch * CHUNK
        pltpu.make_async_copy(ei_hbm.at[pl.ds(E + eoff, CHUNK)], col_v.at[b],
                              isem[b]).wait()
        pltpu.make_async_copy(ei_hbm.at[pl.ds(eoff, CHUNK)], row_v.at[b],
                              isem[b]).wait()
        pltpu.make_async_copy(ew_hbm.at[pl.ds(eoff, CHUNK)], ew_v.at[b],
                              isem[b]).wait()

    def issue_gather(b):
        pltpu.async_copy(s_hbm.at[col_v.at[b]], rows_v.at[b], gsem[b])

    def wait_gather(b):
        pltpu.make_async_copy(s_hbm.at[col_v.at[b]], rows_v.at[b],
                              gsem[b]).wait()

    for b in (0, 1):
        idx_copies(b, b)
    for b in (0, 1):
        wait_idx(b, b)
        issue_gather(b)

    def chunk_step(ch, b):
        wait_gather(b)

        def group_body(g, inner):
            w16 = ew_v[b, pl.ds(g * 16, 16)]
            for t in range(16):
                i = g * 16 + t
                w = jnp.full((16,), w16[t], dtype=jnp.float32)
                for j in range(D // 16):
                    sl = pl.ds(j * 16, 16)
                    rows_v[b, i, sl] = rows_v[b, i, sl] * w
            return inner

        lax.fori_loop(0, CHUNK // 16, group_body, 0)
        pltpu.sync_copy(rows_v.at[b], acc_sh.at[row_v.at[b]], add=True)

        @pl.when(ch + 2 < NCHUNK)
        def _():
            idx_copies(ch + 2, b)
            wait_idx(ch + 2, b)
            issue_gather(b)

    def pair_body(k, carry):
        ch0 = 2 * k
        chunk_step(ch0, 0)

        @pl.when(ch0 + 1 < NCHUNK)
        def _():
            chunk_step(ch0 + 1, 1)

        return carry

    lax.fori_loop(0, (NCHUNK + 1) // 2, pair_body, 0)
    plsc.subcore_barrier()

    pltpu.sync_copy(acc_sh.at[pl.ds(s * RPT, RPT)],
                    out_hbm.at[c, pl.ds(s * RPT, RPT)])


def kernel(x, edge_index, edge_weight, weight_own, weight_nbr, weight_temp,
           bias):
    support = pl.pallas_call(
        _matmul_body,
        out_shape=jax.ShapeDtypeStruct((N, D), jnp.float32),
        grid=(N // ROW_BLOCK,),
        in_specs=[
            pl.BlockSpec((ROW_BLOCK, D), lambda i: (i, 0)),
            pl.BlockSpec((D, D), lambda i: (0, 0)),
            pl.BlockSpec((D, D), lambda i: (0, 0)),
            pl.BlockSpec((D, D), lambda i: (0, 0)),
        ],
        out_specs=pl.BlockSpec((ROW_BLOCK, D), lambda i: (i, 0)),
    )(x, weight_own, weight_nbr, weight_temp)

    ei = edge_index.astype(jnp.int32).reshape(2 * E)
    partials = _make_spmm_kernel()(support, ei, edge_weight)

    out = pl.pallas_call(
        _combine_body,
        out_shape=jax.ShapeDtypeStruct((N, D), jnp.float32),
        grid=(N // ROW_BLOCK,),
        in_specs=[
            pl.BlockSpec((ROW_BLOCK, D), lambda i: (i, 0)),
            pl.BlockSpec((ROW_BLOCK, D), lambda i: (i, 0)),
            pl.BlockSpec((1, D), lambda i: (0, 0)),
        ],
        out_specs=pl.BlockSpec((ROW_BLOCK, D), lambda i: (i, 0)),
    )(partials[0], partials[1], bias.reshape(1, D))
    return out
